# Initial kernel scaffold; baseline (speedup 1.0000x reference)
import jax
import jax.numpy as jnp
from jax.experimental import pallas as pl


def kernel(x):
    raise NotImplementedError("write your pallas kernel here")


# speedup vs baseline: 13.3321x; 13.3321x over previous
import functools
import jax
import jax.numpy as jnp
from jax import lax
from jax.experimental import pallas as pl
from jax.experimental.pallas import tpu as pltpu

NI = 64
CLEAN = 48
D = 256


def _tc_body(x_ref, data_ref, idx_ref):
    S = x_ref[:]
    sim = lax.dot_general(S, S, (((1,), (1,)), ((), ())),
                          preferred_element_type=jnp.float32)

    a = sim[None, :, :]
    b = sim[:, None, :]
    ko = lax.broadcasted_iota(jnp.int32, (NI, NI, NI), 0)
    jo = lax.broadcasted_iota(jnp.int32, (NI, NI, NI), 1)
    cmp = (b < a) | ((b == a) & (ko < jo))
    R = jnp.sum(cmp.astype(jnp.float32), axis=0)

    colsum = jnp.sum(R, axis=1, keepdims=True)

    j_col = lax.broadcasted_iota(jnp.int32, (NI, 1), 0).astype(jnp.float32)
    key = colsum * 64.0 + (63.0 - j_col)

    i_r = lax.broadcasted_iota(jnp.int32, (NI, NI), 0)
    i_c = lax.broadcasted_iota(jnp.int32, (NI, NI), 1)
    eye = (i_r == i_c).astype(jnp.float32)
    key_row = lax.dot_general(key, eye, (((0,), (0,)), ((), ())),
                              precision=lax.Precision.HIGHEST,
                              preferred_element_type=jnp.float32)

    M = key_row > key
    pos = jnp.sum(M.astype(jnp.float32), axis=1, keepdims=True)

    p_row = lax.broadcasted_iota(jnp.int32, (NI, NI), 1).astype(jnp.float32)
    OT = (pos == p_row).astype(jnp.float32)

    gathered = lax.dot_general(OT, S, (((0,), (0,)), ((), ())),
                               precision=lax.Precision.HIGHEST,
                               preferred_element_type=jnp.float32)
    data_ref[:, :] = gathered[:CLEAN, :]

    jj = lax.broadcasted_iota(jnp.int32, (NI, NI), 0).astype(jnp.float32)
    idx_f = jnp.sum(OT * jj, axis=0, keepdims=True)
    idx_ref[0, :, :] = idx_f.astype(jnp.int32)


@jax.jit
def kernel(x):
    B = x.shape[0]
    num_split = B // NI
    data, idx3 = pl.pallas_call(
        _tc_body,
        grid=(num_split,),
        in_specs=[pl.BlockSpec((NI, D), lambda g: (g, 0))],
        out_specs=[
            pl.BlockSpec((CLEAN, D), lambda g: (g, 0)),
            pl.BlockSpec((1, 1, NI), lambda g: (g, 0, 0)),
        ],
        out_shape=[
            jax.ShapeDtypeStruct((num_split * CLEAN, D), jnp.float32),
            jax.ShapeDtypeStruct((num_split, 1, NI), jnp.int32),
        ],
    )(x)
    clean_indices = idx3[:, 0, :CLEAN]
    return (data, clean_indices)


# speedup vs baseline: 23.5370x; 1.7654x over previous
import functools
import jax
import jax.numpy as jnp
from jax import lax
from jax.experimental import pallas as pl
from jax.experimental.pallas import tpu as pltpu

NI = 64
CLEAN = 48
D = 256
GP = 2


def _tc_body(x_ref, data_ref, idx_ref):
    S0 = x_ref[:NI, :]
    S1 = x_ref[NI:, :]
    dn = (((1,), (1,)), ((), ()))
    sim0 = lax.dot_general(S0, S0, dn, preferred_element_type=jnp.float32)
    sim1 = lax.dot_general(S1, S1, dn, preferred_element_type=jnp.float32)
    SIM2 = jnp.concatenate([sim0, sim1], axis=1)

    a = SIM2[None, :, :]
    b = SIM2[:, None, :]
    ko = lax.broadcasted_iota(jnp.int32, (NI, NI, GP * NI), 0)
    jo = lax.broadcasted_iota(jnp.int32, (NI, NI, GP * NI), 1)
    cmp = (b < a) | ((b == a) & (ko < jo))
    R = jnp.sum(cmp.astype(jnp.float32), axis=0)

    gi = lax.broadcasted_iota(jnp.int32, (GP * NI, GP), 0)
    gc = lax.broadcasted_iota(jnp.int32, (GP * NI, GP), 1)
    SEL = ((gi // NI) == gc).astype(jnp.float32)
    colsum = lax.dot_general(R, SEL, (((1,), (0,)), ((), ())),
                             preferred_element_type=jnp.float32)

    j_col = lax.broadcasted_iota(jnp.int32, (NI, GP), 0).astype(jnp.float32)
    key = colsum * 64.0 + (63.0 - j_col)

    i_r = lax.broadcasted_iota(jnp.int32, (NI, NI), 0)
    i_c = lax.broadcasted_iota(jnp.int32, (NI, NI), 1)
    eye = (i_r == i_c).astype(jnp.float32)
    keyT = lax.dot_general(key, eye, (((0,), (0,)), ((), ())),
                           precision=lax.Precision.HIGHEST,
                           preferred_element_type=jnp.float32)

    A = keyT[:, :, None]
    Bm = keyT[:, None, :]
    pos = jnp.sum((Bm > A).astype(jnp.float32), axis=2)

    p_i = lax.broadcasted_iota(jnp.int32, (GP, NI, NI), 2).astype(jnp.float32)
    OT = (pos[:, :, None] == p_i).astype(jnp.float32)

    dn_t = (((0,), (0,)), ((), ()))
    g0 = lax.dot_general(OT[0], S0, dn_t, precision=lax.Precision.HIGHEST,
                         preferred_element_type=jnp.float32)
    g1 = lax.dot_general(OT[1], S1, dn_t, precision=lax.Precision.HIGHEST,
                         preferred_element_type=jnp.float32)
    data_ref[:CLEAN, :] = g0[:CLEAN, :]
    data_ref[CLEAN:, :] = g1[:CLEAN, :]

    jj = lax.broadcasted_iota(jnp.int32, (GP, NI, NI), 1).astype(jnp.float32)
    idx_f = jnp.sum(OT * jj, axis=1)
    idx_ref[0, :, :] = idx_f.astype(jnp.int32)


@jax.jit
def kernel(x):
    B = x.shape[0]
    num_split = B // NI
    steps = num_split // GP
    data, idx3 = pl.pallas_call(
        _tc_body,
        grid=(steps,),
        in_specs=[pl.BlockSpec((GP * NI, D), lambda g: (g, 0))],
        out_specs=[
            pl.BlockSpec((GP * CLEAN, D), lambda g: (g, 0)),
            pl.BlockSpec((1, GP, NI), lambda g: (g, 0, 0)),
        ],
        out_shape=[
            jax.ShapeDtypeStruct((num_split * CLEAN, D), jnp.float32),
            jax.ShapeDtypeStruct((steps, GP, NI), jnp.int32),
        ],
    )(x)
    clean_indices = idx3.reshape(num_split, NI)[:, :CLEAN]
    return (data, clean_indices)


# speedup vs baseline: 25.0620x; 1.0648x over previous
import functools
import jax
import jax.numpy as jnp
from jax import lax
from jax.experimental import pallas as pl
from jax.experimental.pallas import tpu as pltpu

NI = 64
CLEAN = 48
D = 256
GP = 4


def _pair(S0, S1):
    dn = (((1,), (1,)), ((), ()))
    sim0 = lax.dot_general(S0, S0, dn, preferred_element_type=jnp.float32)
    sim1 = lax.dot_general(S1, S1, dn, preferred_element_type=jnp.float32)
    SIM2 = jnp.concatenate([sim0, sim1], axis=1)

    a = SIM2[None, :, :]
    b = SIM2[:, None, :]
    ko = lax.broadcasted_iota(jnp.int32, (NI, NI, 2 * NI), 0)
    jo = lax.broadcasted_iota(jnp.int32, (NI, NI, 2 * NI), 1)
    cmp = (b < a) | ((b == a) & (ko < jo))
    R = jnp.sum(cmp.astype(jnp.float32), axis=0)

    gi = lax.broadcasted_iota(jnp.int32, (2 * NI, 2), 0)
    gc = lax.broadcasted_iota(jnp.int32, (2 * NI, 2), 1)
    SEL = ((gi // NI) == gc).astype(jnp.float32)
    colsum = lax.dot_general(R, SEL, (((1,), (0,)), ((), ())),
                             preferred_element_type=jnp.float32)

    j_col = lax.broadcasted_iota(jnp.int32, (NI, 2), 0).astype(jnp.float32)
    key = colsum * 64.0 + (63.0 - j_col)

    i_r = lax.broadcasted_iota(jnp.int32, (NI, NI), 0)
    i_c = lax.broadcasted_iota(jnp.int32, (NI, NI), 1)
    eye = (i_r == i_c).astype(jnp.float32)
    keyT = lax.dot_general(key, eye, (((0,), (0,)), ((), ())),
                           precision=lax.Precision.HIGHEST,
                           preferred_element_type=jnp.float32)

    A = keyT[:, :, None]
    Bm = keyT[:, None, :]
    pos = jnp.sum((Bm > A).astype(jnp.float32), axis=2)

    p_i = lax.broadcasted_iota(jnp.int32, (2, NI, NI), 2).astype(jnp.float32)
    OT = (pos[:, :, None] == p_i).astype(jnp.float32)

    dn_t = (((0,), (0,)), ((), ()))
    g0 = lax.dot_general(OT[0], S0, dn_t, precision=lax.Precision.HIGHEST,
                         preferred_element_type=jnp.float32)
    g1 = lax.dot_general(OT[1], S1, dn_t, precision=lax.Precision.HIGHEST,
                         preferred_element_type=jnp.float32)
    data = jnp.concatenate([g0[:CLEAN, :], g1[:CLEAN, :]], axis=0)

    jj = lax.broadcasted_iota(jnp.int32, (2, NI, NI), 1).astype(jnp.float32)
    idx_f = jnp.sum(OT * jj, axis=1)
    return data, idx_f.astype(jnp.int32)


def _tc_body(x_ref, data_ref, idx_ref):
    for p in range(GP // 2):
        S0 = x_ref[(2 * p) * NI:(2 * p + 1) * NI, :]
        S1 = x_ref[(2 * p + 1) * NI:(2 * p + 2) * NI, :]
        data, idx = _pair(S0, S1)
        data_ref[(2 * p) * CLEAN:(2 * p + 2) * CLEAN, :] = data
        idx_ref[0, 2 * p:2 * p + 2, :] = idx


@jax.jit
def kernel(x):
    B = x.shape[0]
    num_split = B // NI
    steps = num_split // GP
    data, idx3 = pl.pallas_call(
        _tc_body,
        grid=(steps,),
        in_specs=[pl.BlockSpec((GP * NI, D), lambda g: (g, 0))],
        out_specs=[
            pl.BlockSpec((GP * CLEAN, D), lambda g: (g, 0)),
            pl.BlockSpec((1, GP, NI), lambda g: (g, 0, 0)),
        ],
        out_shape=[
            jax.ShapeDtypeStruct((num_split * CLEAN, D), jnp.float32),
            jax.ShapeDtypeStruct((steps, GP, NI), jnp.int32),
        ],
    )(x)
    clean_indices = idx3.reshape(num_split, NI)[:, :CLEAN]
    return (data, clean_indices)


# speedup vs baseline: 26.3999x; 1.0534x over previous
import functools
import jax
import jax.numpy as jnp
from jax import lax
from jax.experimental import pallas as pl
from jax.experimental.pallas import tpu as pltpu

NI = 64
CLEAN = 48
D = 256
GP = 4


def _pair(S0, S1):
    dn = (((1,), (1,)), ((), ()))
    sim0 = lax.dot_general(S0, S0, dn, preferred_element_type=jnp.float32)
    sim1 = lax.dot_general(S1, S1, dn, preferred_element_type=jnp.float32)
    SIM2 = jnp.concatenate([sim0, sim1], axis=1)

    a = SIM2[None, :, :]
    b = SIM2[:, None, :]
    ko = lax.broadcasted_iota(jnp.int32, (NI, NI, 2 * NI), 0)
    jo = lax.broadcasted_iota(jnp.int32, (NI, NI, 2 * NI), 1)
    cmp = (b < a) | ((b == a) & (ko < jo))
    cmpf = cmp.astype(jnp.float32).reshape(4, 16, NI, 2 * NI)
    R = jnp.sum(jnp.sum(cmpf, axis=1), axis=0)

    gi = lax.broadcasted_iota(jnp.int32, (2 * NI, 2), 0)
    gc = lax.broadcasted_iota(jnp.int32, (2 * NI, 2), 1)
    SEL = ((gi // NI) == gc).astype(jnp.float32)
    colsum = lax.dot_general(R, SEL, (((1,), (0,)), ((), ())),
                             preferred_element_type=jnp.float32)

    j_col = lax.broadcasted_iota(jnp.int32, (NI, 2), 0).astype(jnp.float32)
    key = colsum * 64.0 + (63.0 - j_col)

    i_r = lax.broadcasted_iota(jnp.int32, (NI, NI), 0)
    i_c = lax.broadcasted_iota(jnp.int32, (NI, NI), 1)
    eye = (i_r == i_c).astype(jnp.float32)
    keyT = lax.dot_general(key, eye, (((0,), (0,)), ((), ())),
                           precision=lax.Precision.HIGHEST,
                           preferred_element_type=jnp.float32)

    p_row = lax.broadcasted_iota(jnp.int32, (NI, NI), 1).astype(jnp.float32)
    jj = lax.broadcasted_iota(jnp.int32, (NI, NI), 0).astype(jnp.float32)
    dn_t = (((0,), (0,)), ((), ()))

    outs = []
    idxs = []
    for g, S in ((0, S0), (1, S1)):
        key_col = key[:, g:g + 1]
        key_row = keyT[g:g + 1, :]
        M = (key_row > key_col).astype(jnp.float32)
        pos = jnp.sum(M, axis=1, keepdims=True)
        OT = (pos == p_row).astype(jnp.float32)
        gathered = lax.dot_general(OT, S, dn_t,
                                   precision=lax.Precision.HIGHEST,
                                   preferred_element_type=jnp.float32)
        outs.append(gathered[:CLEAN, :])
        idxs.append(jnp.sum(OT * jj, axis=0, keepdims=True))
    return outs, jnp.concatenate(idxs, axis=0).astype(jnp.int32)


def _tc_body(x_ref, data_ref, idx_ref):
    for p in range(GP // 2):
        S0 = x_ref[(2 * p) * NI:(2 * p + 1) * NI, :]
        S1 = x_ref[(2 * p + 1) * NI:(2 * p + 2) * NI, :]
        (d0, d1), idx = _pair(S0, S1)
        data_ref[(2 * p) * CLEAN:(2 * p + 1) * CLEAN, :] = d0
        data_ref[(2 * p + 1) * CLEAN:(2 * p + 2) * CLEAN, :] = d1
        idx_ref[0, 2 * p:2 * p + 2, :] = idx


@jax.jit
def kernel(x):
    B = x.shape[0]
    num_split = B // NI
    steps = num_split // GP
    data, idx3 = pl.pallas_call(
        _tc_body,
        grid=(steps,),
        in_specs=[pl.BlockSpec((GP * NI, D), lambda g: (g, 0))],
        out_specs=[
            pl.BlockSpec((GP * CLEAN, D), lambda g: (g, 0)),
            pl.BlockSpec((1, GP, NI), lambda g: (g, 0, 0)),
        ],
        out_shape=[
            jax.ShapeDtypeStruct((num_split * CLEAN, D), jnp.float32),
            jax.ShapeDtypeStruct((steps, GP, NI), jnp.int32),
        ],
    )(x)
    clean_indices = idx3.reshape(num_split, NI)[:, :CLEAN]
    return (data, clean_indices)


# speedup vs baseline: 27.4794x; 1.0409x over previous
import functools
import jax
import jax.numpy as jnp
from jax import lax
from jax.experimental import pallas as pl
from jax.experimental.pallas import tpu as pltpu

NI = 64
CLEAN = 48
D = 256
GP = 8


def _pair(S0, S1):
    dn = (((1,), (1,)), ((), ()))
    sim0 = lax.dot_general(S0, S0, dn, preferred_element_type=jnp.float32)
    sim1 = lax.dot_general(S1, S1, dn, preferred_element_type=jnp.float32)
    SIM2 = jnp.concatenate([sim0, sim1], axis=1)

    a = SIM2[None, :, :]
    b = SIM2[:, None, :]
    ko = lax.broadcasted_iota(jnp.int32, (NI, NI, 2 * NI), 0)
    jo = lax.broadcasted_iota(jnp.int32, (NI, NI, 2 * NI), 1)
    cmp = (b < a) | ((b == a) & (ko < jo))
    cmpf = cmp.astype(jnp.float32).reshape(4, 16, NI, 2 * NI)
    R = jnp.sum(jnp.sum(cmpf, axis=1), axis=0)

    gi = lax.broadcasted_iota(jnp.int32, (2 * NI, 2), 0)
    gc = lax.broadcasted_iota(jnp.int32, (2 * NI, 2), 1)
    SEL = ((gi // NI) == gc).astype(jnp.float32)
    colsum = lax.dot_general(R, SEL, (((1,), (0,)), ((), ())),
                             preferred_element_type=jnp.float32)

    j_col = lax.broadcasted_iota(jnp.int32, (NI, 2), 0).astype(jnp.float32)
    key = colsum * 64.0 + (63.0 - j_col)

    i_r = lax.broadcasted_iota(jnp.int32, (NI, NI), 0)
    i_c = lax.broadcasted_iota(jnp.int32, (NI, NI), 1)
    eye = (i_r == i_c).astype(jnp.float32)
    keyT = lax.dot_general(key, eye, (((0,), (0,)), ((), ())),
                           precision=lax.Precision.HIGHEST,
                           preferred_element_type=jnp.float32)

    p_row = lax.broadcasted_iota(jnp.int32, (NI, NI), 1).astype(jnp.float32)
    jj = lax.broadcasted_iota(jnp.int32, (NI, NI), 0).astype(jnp.float32)
    dn_t = (((0,), (0,)), ((), ()))

    outs = []
    idxs = []
    for g, S in ((0, S0), (1, S1)):
        key_col = key[:, g:g + 1]
        key_row = keyT[g:g + 1, :]
        M = (key_row > key_col).astype(jnp.float32)
        pos = jnp.sum(M, axis=1, keepdims=True)
        OT = (pos == p_row).astype(jnp.float32)
        gathered = lax.dot_general(OT, S, dn_t,
                                   precision=lax.Precision.HIGHEST,
                                   preferred_element_type=jnp.float32)
        outs.append(gathered[:CLEAN, :])
        idxs.append(jnp.sum(OT * jj, axis=0, keepdims=True))
    return outs, jnp.concatenate(idxs, axis=0).astype(jnp.int32)


def _tc_body(x_ref, data_ref, idx_ref):
    for p in range(GP // 2):
        S0 = x_ref[(2 * p) * NI:(2 * p + 1) * NI, :]
        S1 = x_ref[(2 * p + 1) * NI:(2 * p + 2) * NI, :]
        (d0, d1), idx = _pair(S0, S1)
        data_ref[(2 * p) * CLEAN:(2 * p + 1) * CLEAN, :] = d0
        data_ref[(2 * p + 1) * CLEAN:(2 * p + 2) * CLEAN, :] = d1
        idx_ref[0, 2 * p:2 * p + 2, :] = idx


@jax.jit
def kernel(x):
    B = x.shape[0]
    num_split = B // NI
    steps = num_split // GP
    data, idx3 = pl.pallas_call(
        _tc_body,
        grid=(steps,),
        in_specs=[pl.BlockSpec((GP * NI, D), lambda g: (g, 0))],
        out_specs=[
            pl.BlockSpec((GP * CLEAN, D), lambda g: (g, 0)),
            pl.BlockSpec((1, GP, NI), lambda g: (g, 0, 0)),
        ],
        out_shape=[
            jax.ShapeDtypeStruct((num_split * CLEAN, D), jnp.float32),
            jax.ShapeDtypeStruct((steps, GP, NI), jnp.int32),
        ],
    )(x)
    clean_indices = idx3.reshape(num_split, NI)[:, :CLEAN]
    return (data, clean_indices)


# speedup vs baseline: 34.4495x; 1.2536x over previous
import functools
import jax
import jax.numpy as jnp
from jax import lax
from jax.experimental import pallas as pl
from jax.experimental.pallas import tpu as pltpu
from jax.experimental.pallas import tpu_sc as plsc

NI = 64
CLEAN = 48
D = 256
GP = 8

SC_CORES = 2
SC_SUBCORES = 16
NW = SC_CORES * SC_SUBCORES


def _pair_idx(S0, S1):
    dn = (((1,), (1,)), ((), ()))
    sim0 = lax.dot_general(S0, S0, dn, preferred_element_type=jnp.float32)
    sim1 = lax.dot_general(S1, S1, dn, preferred_element_type=jnp.float32)
    SIM2 = jnp.concatenate([sim0, sim1], axis=1)

    a = SIM2[None, :, :]
    b = SIM2[:, None, :]
    ko = lax.broadcasted_iota(jnp.int32, (NI, NI, 2 * NI), 0)
    jo = lax.broadcasted_iota(jnp.int32, (NI, NI, 2 * NI), 1)
    cmp = (b < a) | ((b == a) & (ko < jo))
    cmpf = cmp.astype(jnp.float32).reshape(4, 16, NI, 2 * NI)
    R = jnp.sum(jnp.sum(cmpf, axis=1), axis=0)

    gi = lax.broadcasted_iota(jnp.int32, (2 * NI, 2), 0)
    gc = lax.broadcasted_iota(jnp.int32, (2 * NI, 2), 1)
    SEL = ((gi // NI) == gc).astype(jnp.float32)
    colsum = lax.dot_general(R, SEL, (((1,), (0,)), ((), ())),
                             preferred_element_type=jnp.float32)

    j_col = lax.broadcasted_iota(jnp.int32, (NI, 2), 0).astype(jnp.float32)
    key = colsum * 64.0 + (63.0 - j_col)

    i_r = lax.broadcasted_iota(jnp.int32, (NI, NI), 0)
    i_c = lax.broadcasted_iota(jnp.int32, (NI, NI), 1)
    eye = (i_r == i_c).astype(jnp.float32)
    keyT = lax.dot_general(key, eye, (((0,), (0,)), ((), ())),
                           precision=lax.Precision.HIGHEST,
                           preferred_element_type=jnp.float32)

    p_row = lax.broadcasted_iota(jnp.int32, (NI, NI), 1).astype(jnp.float32)
    jj = lax.broadcasted_iota(jnp.int32, (NI, NI), 0).astype(jnp.float32)

    idxs = []
    for g in (0, 1):
        key_col = key[:, g:g + 1]
        key_row = keyT[g:g + 1, :]
        M = (key_row > key_col).astype(jnp.float32)
        pos = jnp.sum(M, axis=1, keepdims=True)
        OT = (pos == p_row).astype(jnp.float32)
        idxs.append(jnp.sum(OT * jj, axis=0, keepdims=True))
    return jnp.concatenate(idxs, axis=0).astype(jnp.int32)


def _tc_body(x_ref, idx_ref):
    pid = pl.program_id(0)
    for p in range(GP // 2):
        S0 = x_ref[(2 * p) * NI:(2 * p + 1) * NI, :]
        S1 = x_ref[(2 * p + 1) * NI:(2 * p + 2) * NI, :]
        idx = _pair_idx(S0, S1)
        base = (pid * GP + 2 * p) * NI
        gof = lax.broadcasted_iota(jnp.int32, (2, NI), 0) * NI
        idx_ref[0, 2 * p:2 * p + 2, :] = idx + gof + base


def _tc_indices(x, num_split):
    steps = num_split // GP
    idx3 = pl.pallas_call(
        _tc_body,
        grid=(steps,),
        in_specs=[pl.BlockSpec((GP * NI, D), lambda g: (g, 0))],
        out_specs=pl.BlockSpec((1, GP, NI), lambda g: (g, 0, 0)),
        out_shape=jax.ShapeDtypeStruct((steps, GP, NI), jnp.int32),
    )(x)
    return idx3.reshape(num_split, NI)


def _sc_gather(x, flat_idx, n_rows):
    bpw = n_rows // NW
    mesh = plsc.VectorSubcoreMesh(core_axis_name="c", subcore_axis_name="s")

    @functools.partial(
        pl.kernel, mesh=mesh,
        out_type=jax.ShapeDtypeStruct((n_rows, D), jnp.float32),
        scratch_types=[
            pltpu.VMEM((bpw,), jnp.int32),
            pltpu.VMEM((bpw, D), jnp.float32),
            pltpu.SemaphoreType.DMA,
        ],
    )
    def k(x_hbm, idx_hbm, out_hbm, idx_v, rows_v, sem):
        wid = lax.axis_index("s") * SC_CORES + lax.axis_index("c")
        base = wid * bpw
        pltpu.sync_copy(idx_hbm.at[pl.ds(base, bpw)], idx_v)
        pltpu.async_copy(x_hbm.at[idx_v], rows_v, sem).wait()
        pltpu.sync_copy(rows_v, out_hbm.at[pl.ds(base, bpw)])

    return k(x, flat_idx)


@jax.jit
def kernel(x):
    B = x.shape[0]
    num_split = B // NI
    glob = _tc_indices(x, num_split)
    glob48 = glob[:, :CLEAN]
    flat_idx = glob48.reshape(-1)
    clean_data = _sc_gather(x, flat_idx, num_split * CLEAN)
    clean_indices = glob48 - NI * jnp.arange(num_split, dtype=jnp.int32)[:, None]
    return (clean_data, clean_indices)
